# baseline (speedup 1.0000x reference)
import functools

import jax
import jax.numpy as jnp
from jax import lax
from jax.experimental import pallas as pl
from jax.experimental.pallas import tpu as pltpu
from jax.experimental.pallas import tpu_sc as plsc

N = 10000
E = 320000
NC = 2
NS = 16
NW = NC * NS
EPW = E // NW
EPB = 128
EPT = 10240
N_PAD = 10240
RPT = N_PAD // NS
NB = 8


def _make_seg_sum(width):
  n_chunks = EPT // EPB
  H = NB // 2
  n_outer = n_chunks // NB - 2
  mesh = plsc.VectorSubcoreMesh(core_axis_name="c", subcore_axis_name="s")

  @functools.partial(
      pl.kernel,
      out_type=jax.ShapeDtypeStruct((NC, N_PAD, width), jnp.float32),
      mesh=mesh,
      compiler_params=pltpu.CompilerParams(use_tc_tiling_on_sc=False),
      scratch_types=[
          pltpu.VMEM((EPT,), jnp.int32),
          pltpu.VMEM((EPT,), jnp.int32),
          pltpu.VMEM((NB, EPB, width), jnp.float32),
          pltpu.VMEM((EPB, width), jnp.float32),
          pltpu.VMEM_SHARED((N_PAD, width), jnp.float32),
          pltpu.SemaphoreType.DMA((NB,)),
          pltpu.SemaphoreType.DMA((NB,)),
      ],
  )
  def seg(m_hbm, edge_hbm, out_hbm,
          src_v, dst_v, rows_v, zero_v, acc_sh, gsems, ssems):
    c = lax.axis_index("c")
    s = lax.axis_index("s")
    wid = c * NS + s

    def gather(j, b):
      return pltpu.make_async_copy(
          m_hbm.at[src_v.at[pl.ds(j * EPB, EPB)]], rows_v.at[b], gsems.at[b])

    def scat(j, b):
      return pltpu.make_async_copy(
          rows_v.at[b], acc_sh.at[dst_v.at[pl.ds(j * EPB, EPB)]], ssems.at[b])

    pltpu.sync_copy(edge_hbm.at[0, pl.ds(wid * EPW, EPW)],
                    src_v.at[pl.ds(0, EPW)])
    pltpu.sync_copy(edge_hbm.at[1, pl.ds(wid * EPW, EPW)],
                    dst_v.at[pl.ds(0, EPW)])
    lane = lax.iota(jnp.int32, 16)
    for t in range((EPT - EPW) // 16):
      src_v[pl.ds(EPW + t * 16, 16)] = lane + t * 16
      dst_v[pl.ds(EPW + t * 16, 16)] = lane + (N + t * 16)
    for t in range(H):
      gather(t, t).start()

    z = jnp.zeros((16,), jnp.float32)
    for r in range(EPB):
      for q in range(width // 16):
        zero_v[r, pl.ds(q * 16, 16)] = z
    for t in range(RPT // EPB):
      pltpu.sync_copy(zero_v, acc_sh.at[pl.ds(s * RPT + t * EPB, EPB)])
    plsc.subcore_barrier()

    for t in range(NB):
      gather(t, t).wait()
      scat(t, t).start(add=True)
      t2 = (t + H) % NB
      if t >= H:
        scat(t - H, t2).wait()
      gather(t + H, t2).start()

    def body(k, carry):
      base = (k + 1) * NB
      for t in range(NB):
        j = base + t
        t2 = (t + H) % NB
        gather(j, t).wait()
        scat(j, t).start(add=True)
        scat(j - H, t2).wait()
        gather(j + H, t2).start()
      return carry

    lax.fori_loop(0, n_outer, body, 0)

    j0 = (n_outer + 1) * NB
    for t in range(NB):
      j = j0 + t
      t2 = (t + H) % NB
      gather(j, t).wait()
      scat(j, t).start(add=True)
      scat(j - H, t2).wait()
      if t < H:
        gather(j + H, t2).start()
    for t in range(H):
      scat(j0 + H + t, H + t).wait()
    plsc.subcore_barrier()
    pltpu.sync_copy(acc_sh.at[pl.ds(s * RPT, RPT)],
                    out_hbm.at[c, pl.ds(s * RPT, RPT)])

  return seg


_seg32 = _make_seg_sum(32)
_seg16 = _make_seg_sum(16)


def _tc1_body(x_ref, w1a1_ref, w1a2_ref, w1b1_ref, w1b2_ref, m_ref, s_ref):
  zr = jnp.zeros((1, 16), jnp.float32)
  w = jnp.concatenate([
      jnp.concatenate([w1a1_ref[...], zr], axis=0),
      jnp.concatenate([w1b1_ref[...], zr], axis=0),
      jnp.concatenate([w1a2_ref[...], zr], axis=0),
      jnp.concatenate([w1b2_ref[...], zr], axis=0)], axis=1)
  acc = jnp.dot(x_ref[...], w, preferred_element_type=jnp.float32)
  m_ref[...] = acc[:, :32]
  s_ref[...] = acc[:, 32:]


def _elu(v):
  return jnp.where(v > 0.0, v, jnp.exp(v) - 1.0)


def _tc2_body(a_ref, s_ref, b1a_ref, b1b_ref, w21_ref, w22_ref, b2_ref,
              p_ref, s2_ref):
  b1 = jnp.concatenate([b1a_ref[...].reshape(1, 16),
                        b1b_ref[...].reshape(1, 16)], axis=1)
  z = a_ref[0, :N] + a_ref[1, :N] + s_ref[...] + b1
  h = _elu(0.5 * (_elu(z[:, :16]) + _elu(z[:, 16:])))
  p7 = jnp.dot(h, w21_ref[...], preferred_element_type=jnp.float32)
  p_ref[...] = jnp.concatenate([p7, jnp.zeros((N, 9), jnp.float32)], axis=1)
  s2_ref[...] = (jnp.dot(h, w22_ref[...], preferred_element_type=jnp.float32)
                 + b2_ref[...].reshape(1, 7))


def _tc3_body(a_ref, s2_ref, o_ref):
  z = (a_ref[0, :N] + a_ref[1, :N])[:, :7] + s2_ref[...]
  z = z - jnp.max(z, axis=1, keepdims=True)
  e = jnp.exp(z)
  o_ref[...] = e / jnp.sum(e, axis=1, keepdims=True)


def kernel(x, w1a1, w1a2, b1a, w1b1, w1b2, b1b, w21, w22, b2, edge_index):
  f32 = jnp.float32
  full = lambda shape: pl.BlockSpec(shape, lambda: (0,) * len(shape))

  m1, s1 = pl.pallas_call(
      _tc1_body,
      in_specs=[full((N, 128)), full((127, 16)), full((127, 16)),
                full((127, 16)), full((127, 16))],
      out_specs=[full((N, 32)), full((N, 32))],
      out_shape=[jax.ShapeDtypeStruct((N, 32), f32),
                 jax.ShapeDtypeStruct((N, 32), f32)],
  )(x, w1a1, w1a2, w1b1, w1b2)

  parts1 = _seg32(m1, edge_index)

  p2, s2 = pl.pallas_call(
      _tc2_body,
      in_specs=[full((NC, N_PAD, 32)), full((N, 32)), full((16,)),
                full((16,)), full((16, 7)), full((16, 7)), full((7,))],
      out_specs=[full((N, 16)), full((N, 7))],
      out_shape=[jax.ShapeDtypeStruct((N, 16), f32),
                 jax.ShapeDtypeStruct((N, 7), f32)],
  )(parts1, s1, b1a, b1b, w21, w22, b2)

  parts2 = _seg16(p2, edge_index)

  return pl.pallas_call(
      _tc3_body,
      in_specs=[full((NC, N_PAD, 16)), full((N, 7))],
      out_specs=full((N, 7)),
      out_shape=jax.ShapeDtypeStruct((N, 7), f32),
  )(parts2, s2)


# speedup vs baseline: 1.0285x; 1.0285x over previous
import functools

import jax
import jax.numpy as jnp
from jax import lax
from jax.experimental import pallas as pl
from jax.experimental.pallas import tpu as pltpu
from jax.experimental.pallas import tpu_sc as plsc

N = 10000
E = 320000
NC = 2
NS = 16
NW = NC * NS
EPW = E // NW
EPB = 256
EPT = 10240
N_PAD = 10240
RPT = N_PAD // NS
NB = 8
ZB = 128


def _make_seg_sum(width):
  n_chunks = EPT // EPB
  n_outer = n_chunks - NB
  mesh = plsc.VectorSubcoreMesh(core_axis_name="c", subcore_axis_name="s")

  @functools.partial(
      pl.kernel,
      out_type=jax.ShapeDtypeStruct((NC, N_PAD, width), jnp.float32),
      mesh=mesh,
      compiler_params=pltpu.CompilerParams(use_tc_tiling_on_sc=False),
      scratch_types=[
          pltpu.VMEM((EPT,), jnp.int32),
          pltpu.VMEM((EPT,), jnp.int32),
          pltpu.VMEM((NB, EPB, width), jnp.float32),
          pltpu.VMEM((ZB, width), jnp.float32),
          pltpu.VMEM_SHARED((N_PAD, width), jnp.float32),
          pltpu.SemaphoreType.DMA((NB,)),
      ],
  )
  def seg(m_hbm, edge_hbm, out_hbm,
          src_v, dst_v, rows_v, zero_v, acc_sh, gsems):
    c = lax.axis_index("c")
    s = lax.axis_index("s")
    wid = c * NS + s

    def gather(j, b):
      return pltpu.make_async_copy(
          m_hbm.at[src_v.at[pl.ds(j * EPB, EPB)]], rows_v.at[b], gsems.at[b])

    pltpu.sync_copy(edge_hbm.at[0, pl.ds(wid * EPW, EPW)],
                    src_v.at[pl.ds(0, EPW)])
    pltpu.sync_copy(edge_hbm.at[1, pl.ds(wid * EPW, EPW)],
                    dst_v.at[pl.ds(0, EPW)])
    lane = lax.iota(jnp.int32, 16)
    for t in range((EPT - EPW) // 16):
      src_v[pl.ds(EPW + t * 16, 16)] = lane + t * 16
      dst_v[pl.ds(EPW + t * 16, 16)] = lane + (N + t * 16)
    for b in range(NB):
      gather(b, b).start()

    z = jnp.zeros((16,), jnp.float32)
    for r in range(ZB):
      for q in range(width // 16):
        zero_v[r, pl.ds(q * 16, 16)] = z
    for t in range(RPT // ZB):
      pltpu.sync_copy(zero_v, acc_sh.at[pl.ds(s * RPT + t * ZB, ZB)])
    plsc.subcore_barrier()

    def body(i, carry):
      for b in range(NB):
        j = i * NB + b
        gather(j, b).wait()
        pltpu.sync_copy(rows_v.at[b],
                        acc_sh.at[dst_v.at[pl.ds(j * EPB, EPB)]], add=True)
        gather(j + NB, b).start()
      return carry

    lax.fori_loop(0, n_outer // NB, body, 0)
    for b in range(NB):
      j = n_outer + b
      gather(j, b).wait()
      pltpu.sync_copy(rows_v.at[b],
                      acc_sh.at[dst_v.at[pl.ds(j * EPB, EPB)]], add=True)
    plsc.subcore_barrier()
    pltpu.sync_copy(acc_sh.at[pl.ds(s * RPT, RPT)],
                    out_hbm.at[c, pl.ds(s * RPT, RPT)])

  return seg


_seg32 = _make_seg_sum(32)
_seg16 = _make_seg_sum(16)


def _tc1_body(x_ref, w1a1_ref, w1a2_ref, w1b1_ref, w1b2_ref, m_ref, s_ref):
  zr = jnp.zeros((1, 16), jnp.float32)
  w = jnp.concatenate([
      jnp.concatenate([w1a1_ref[...], zr], axis=0),
      jnp.concatenate([w1b1_ref[...], zr], axis=0),
      jnp.concatenate([w1a2_ref[...], zr], axis=0),
      jnp.concatenate([w1b2_ref[...], zr], axis=0)], axis=1)
  acc = jnp.dot(x_ref[...], w, preferred_element_type=jnp.float32)
  m_ref[...] = acc[:, :32]
  s_ref[...] = acc[:, 32:]


def _elu(v):
  return jnp.where(v > 0.0, v, jnp.exp(v) - 1.0)


def _tc2_body(a_ref, s_ref, b1a_ref, b1b_ref, w21_ref, w22_ref, b2_ref,
              p_ref, s2_ref):
  b1 = jnp.concatenate([b1a_ref[...].reshape(1, 16),
                        b1b_ref[...].reshape(1, 16)], axis=1)
  z = a_ref[0] + a_ref[1] + s_ref[...] + b1
  h = _elu(0.5 * (_elu(z[:, :16]) + _elu(z[:, 16:])))
  p7 = jnp.dot(h, w21_ref[...], preferred_element_type=jnp.float32)
  p_ref[...] = jnp.concatenate(
      [p7, jnp.zeros(p7.shape[:1] + (9,), jnp.float32)], axis=1)
  s2_ref[...] = (jnp.dot(h, w22_ref[...], preferred_element_type=jnp.float32)
                 + b2_ref[...].reshape(1, 7))


def _tc3_body(a_ref, s2_ref, o_ref):
  z = (a_ref[0] + a_ref[1])[:, :7] + s2_ref[...]
  z = z - jnp.max(z, axis=1, keepdims=True)
  e = jnp.exp(z)
  o_ref[...] = e / jnp.sum(e, axis=1, keepdims=True)


BM = 1280
NBLK = N_PAD // BM


def kernel(x, w1a1, w1a2, b1a, w1b1, w1b2, b1b, w21, w22, b2, edge_index):
  f32 = jnp.float32
  full = lambda shape: pl.BlockSpec(shape, lambda i: (0,) * len(shape))
  rows = lambda width: pl.BlockSpec((BM, width), lambda i: (i, 0))
  rows3 = lambda width: pl.BlockSpec((NC, BM, width), lambda i: (0, i, 0))

  m1, s1 = pl.pallas_call(
      _tc1_body,
      grid=(NBLK,),
      in_specs=[rows(128), full((127, 16)), full((127, 16)),
                full((127, 16)), full((127, 16))],
      out_specs=[rows(32), rows(32)],
      out_shape=[jax.ShapeDtypeStruct((N, 32), f32),
                 jax.ShapeDtypeStruct((N, 32), f32)],
  )(x, w1a1, w1a2, w1b1, w1b2)

  parts1 = _seg32(m1, edge_index)

  p2, s2 = pl.pallas_call(
      _tc2_body,
      grid=(NBLK,),
      in_specs=[rows3(32), rows(32), full((16,)),
                full((16,)), full((16, 7)), full((16, 7)), full((7,))],
      out_specs=[rows(16), rows(7)],
      out_shape=[jax.ShapeDtypeStruct((N, 16), f32),
                 jax.ShapeDtypeStruct((N, 7), f32)],
  )(parts1, s1, b1a, b1b, w21, w22, b2)

  parts2 = _seg16(p2, edge_index)

  return pl.pallas_call(
      _tc3_body,
      grid=(NBLK,),
      in_specs=[rows3(16), rows(7)],
      out_specs=rows(7),
      out_shape=jax.ShapeDtypeStruct((N, 7), f32),
  )(parts2, s2)


# speedup vs baseline: 1.1065x; 1.0758x over previous
import functools

import jax
import jax.numpy as jnp
from jax import lax
from jax.experimental import pallas as pl
from jax.experimental.pallas import tpu as pltpu
from jax.experimental.pallas import tpu_sc as plsc

N = 10000
E = 320000
NC = 2
NS = 16
NW = NC * NS
EPW = E // NW
EPB = 128
EPT = 10240
N_PAD = 10240
RPT = N_PAD // NS
NB = 8
ZB = 128


def _make_seg_sum(width):
  n_chunks = EPT // EPB
  n_outer = n_chunks - NB
  mesh = plsc.VectorSubcoreMesh(core_axis_name="c", subcore_axis_name="s")

  @functools.partial(
      pl.kernel,
      out_type=jax.ShapeDtypeStruct((NC, N_PAD, width), jnp.float32),
      mesh=mesh,
      compiler_params=pltpu.CompilerParams(use_tc_tiling_on_sc=False),
      scratch_types=[
          pltpu.VMEM((EPT,), jnp.int32),
          pltpu.VMEM((EPT,), jnp.int32),
          pltpu.VMEM((NB, EPB, width), jnp.float32),
          pltpu.VMEM((ZB, width), jnp.float32),
          pltpu.VMEM_SHARED((N_PAD, width), jnp.float32),
          pltpu.SemaphoreType.DMA((NB,)),
      ],
  )
  def seg(m_hbm, edge_hbm, out_hbm,
          src_v, dst_v, rows_v, zero_v, acc_sh, gsems):
    c = lax.axis_index("c")
    s = lax.axis_index("s")
    wid = c * NS + s

    def gather(j, b):
      return pltpu.make_async_copy(
          m_hbm.at[src_v.at[pl.ds(j * EPB, EPB)]], rows_v.at[b], gsems.at[b])

    pltpu.sync_copy(edge_hbm.at[0, pl.ds(wid * EPW, EPW)],
                    src_v.at[pl.ds(0, EPW)])
    pltpu.sync_copy(edge_hbm.at[1, pl.ds(wid * EPW, EPW)],
                    dst_v.at[pl.ds(0, EPW)])
    lane = lax.iota(jnp.int32, 16)
    for t in range((EPT - EPW) // 16):
      src_v[pl.ds(EPW + t * 16, 16)] = lane + t * 16
      dst_v[pl.ds(EPW + t * 16, 16)] = lane + (N + t * 16)
    for b in range(NB):
      gather(b, b).start()

    z = jnp.zeros((16,), jnp.float32)
    for r in range(ZB):
      for q in range(width // 16):
        zero_v[r, pl.ds(q * 16, 16)] = z
    for t in range(RPT // ZB):
      pltpu.sync_copy(zero_v, acc_sh.at[pl.ds(s * RPT + t * ZB, ZB)])
    plsc.subcore_barrier()

    def body(i, carry):
      for b in range(NB):
        j = i * NB + b
        gather(j, b).wait()
        pltpu.sync_copy(rows_v.at[b],
                        acc_sh.at[dst_v.at[pl.ds(j * EPB, EPB)]], add=True)
        gather(j + NB, b).start()
      return carry

    lax.fori_loop(0, n_outer // NB, body, 0)
    for b in range(NB):
      j = n_outer + b
      gather(j, b).wait()
      pltpu.sync_copy(rows_v.at[b],
                      acc_sh.at[dst_v.at[pl.ds(j * EPB, EPB)]], add=True)
    plsc.subcore_barrier()
    pltpu.sync_copy(acc_sh.at[pl.ds(s * RPT, RPT)],
                    out_hbm.at[c, pl.ds(s * RPT, RPT)])

  return seg


_seg32 = _make_seg_sum(32)
_seg16 = _make_seg_sum(16)


def _tc1_body(x_ref, w_ref, m_ref, s_ref):
  zr = jnp.zeros((1, 16), jnp.float32)
  w = jnp.concatenate([
      jnp.concatenate([w_ref[0], zr], axis=0),
      jnp.concatenate([w_ref[1], zr], axis=0),
      jnp.concatenate([w_ref[2], zr], axis=0),
      jnp.concatenate([w_ref[3], zr], axis=0)], axis=1)
  acc = jnp.dot(x_ref[...], w, preferred_element_type=jnp.float32)
  m_ref[...] = acc[:, :32]
  s_ref[...] = acc[:, 32:]


def _elu(v):
  return jnp.where(v > 0.0, v, jnp.exp(v) - 1.0)


def _tc2_body(a_ref, s_ref, b1a_ref, b1b_ref, w21_ref, w22_ref, b2_ref,
              p_ref, s2_ref):
  b1 = jnp.concatenate([b1a_ref[...].reshape(1, 16),
                        b1b_ref[...].reshape(1, 16)], axis=1)
  z = a_ref[0, :N] + a_ref[1, :N] + s_ref[...] + b1
  h = _elu(0.5 * (_elu(z[:, :16]) + _elu(z[:, 16:])))
  p7 = jnp.dot(h, w21_ref[...], preferred_element_type=jnp.float32)
  p_ref[...] = jnp.concatenate(
      [p7, jnp.zeros(p7.shape[:1] + (9,), jnp.float32)], axis=1)
  s2_ref[...] = (jnp.dot(h, w22_ref[...], preferred_element_type=jnp.float32)
                 + b2_ref[...].reshape(1, 7))


def _tc3_body(a_ref, s2_ref, o_ref):
  z = (a_ref[0, :N] + a_ref[1, :N])[:, :7] + s2_ref[...]
  z = z - jnp.max(z, axis=1, keepdims=True)
  e = jnp.exp(z)
  o_ref[...] = e / jnp.sum(e, axis=1, keepdims=True)


def kernel(x, w1a1, w1a2, b1a, w1b1, w1b2, b1b, w21, w22, b2, edge_index):
  f32 = jnp.float32
  full = lambda shape: pl.BlockSpec(shape, lambda: (0,) * len(shape))
  w_stack = jnp.stack([w1a1, w1b1, w1a2, w1b2])

  m1, s1 = pl.pallas_call(
      _tc1_body,
      in_specs=[full((N, 128)), full((4, 127, 16))],
      out_specs=[full((N, 32)), full((N, 32))],
      out_shape=[jax.ShapeDtypeStruct((N, 32), f32),
                 jax.ShapeDtypeStruct((N, 32), f32)],
  )(x, w_stack)

  parts1 = _seg32(m1, edge_index)

  p2, s2 = pl.pallas_call(
      _tc2_body,
      in_specs=[full((NC, N_PAD, 32)), full((N, 32)), full((16,)),
                full((16,)), full((16, 7)), full((16, 7)), full((7,))],
      out_specs=[full((N, 16)), full((N, 7))],
      out_shape=[jax.ShapeDtypeStruct((N, 16), f32),
                 jax.ShapeDtypeStruct((N, 7), f32)],
  )(parts1, s1, b1a, b1b, w21, w22, b2)

  parts2 = _seg16(p2, edge_index)

  return pl.pallas_call(
      _tc3_body,
      in_specs=[full((NC, N_PAD, 16)), full((N, 7))],
      out_specs=full((N, 7)),
      out_shape=jax.ShapeDtypeStruct((N, 7), f32),
  )(parts2, s2)
